# Initial kernel scaffold; baseline (speedup 1.0000x reference)
import jax
import jax.numpy as jnp
from jax.experimental import pallas as pl


def kernel(x, codebooks):
    raise NotImplementedError("write your pallas kernel here")


# speedup vs baseline: 1.3530x; 1.3530x over previous
import functools

import jax
import jax.numpy as jnp
from jax import lax
from jax.experimental import pallas as pl
from jax.experimental.pallas import tpu as pltpu
from jax.experimental.pallas import tpu_sc as plsc

NUM_Q = 8
K = 8192
D = 32
B = 16
N = 1024
T = B * N
TBLK = 256
GRID = T // TBLK
KC = 2048
NKC = K // KC

DP = 128

NC, NS = 2, 16
NW = NC * NS
TPW = T // NW
CHUNK = 128
NCHUNK = TPW // CHUNK
L = 16


def _layer_body(r_ref, r2_ref, cbt_ref, ind_out, stat_out):
    rr = r_ref[...]
    r2 = r2_ref[...]
    run_val = jnp.full((TBLK, 128), jnp.inf, dtype=jnp.float32)
    run_idx = jnp.zeros((TBLK, 128), dtype=jnp.int32)
    for kc in range(NKC):
        cbt = cbt_ref[:, kc * KC:(kc + 1) * KC]
        c2 = jnp.sum(cbt * cbt, axis=0, keepdims=True)
        m = lax.dot_general(rr, cbt, (((1,), (0,)), ((), ())),
                            preferred_element_type=jnp.float32)
        val = (r2 - (m + m)) + c2
        idx = lax.broadcasted_iota(jnp.int32, (TBLK, KC), 1) + kc * KC
        w = KC
        while w > 128:
            w //= 2
            v1, v2 = val[:, :w], val[:, w:]
            i1, i2 = idx[:, :w], idx[:, w:]
            take = v2 < v1
            val = jnp.where(take, v2, v1)
            idx = jnp.where(take, i2, i1)
        take = val < run_val
        run_val = jnp.where(take, val, run_val)
        run_idx = jnp.where(take, idx, run_idx)

    minv = jnp.min(run_val, axis=1, keepdims=True)
    ind = jnp.min(jnp.where(run_val <= minv, run_idx, K), axis=1)
    ind_out[...] = ind.reshape(1, 1, TBLK)

    partial = jnp.sum(minv)

    @pl.when(pl.program_id(0) == 0)
    def _():
        stat_out[...] = jnp.zeros_like(stat_out)

    stat_out[...] += partial


_TOK_SPEC = pl.BlockSpec((TBLK, D), lambda i: (i, 0))
_R2_SPEC = pl.BlockSpec((TBLK, 1), lambda i: (i, 0))
_CBT_SPEC = pl.BlockSpec((D, K), lambda i: (0, 0))
_IND_SPEC = pl.BlockSpec((1, 1, TBLK), lambda i: (i, 0, 0))
_STAT_SPEC = pl.BlockSpec((8, 128), lambda i: (0, 0))
_IND_SHAPE = jax.ShapeDtypeStruct((GRID, 1, TBLK), jnp.int32)
_STAT_SHAPE = jax.ShapeDtypeStruct((8, 128), jnp.float32)

_layer_call = pl.pallas_call(
    _layer_body,
    grid=(GRID,),
    in_specs=[_TOK_SPEC, _R2_SPEC, _CBT_SPEC],
    out_specs=[_IND_SPEC, _STAT_SPEC],
    out_shape=[_IND_SHAPE, _STAT_SHAPE],
)


def _sc_body(final, cb_hbm, idx_hbm, r_hbm, x_hbm, o_hbm,
             idx_v, rows_v, r_v, x_v, o_v, sem):
    wid = lax.axis_index("s") * NC + lax.axis_index("c")
    pltpu.sync_copy(idx_hbm.at[wid], idx_v)
    for j in range(NCHUNK):
        base = (wid * NCHUNK + j) * CHUNK
        cp = pltpu.async_copy(cb_hbm.at[idx_v.at[j]], rows_v, sem)
        pltpu.sync_copy(r_hbm.at[pl.ds(base, CHUNK)], r_v)
        if final:
            pltpu.sync_copy(x_hbm.at[pl.ds(base, CHUNK)], x_v)
        cp.wait()

        def _tok(t, _):
            for h in range(D // L):
                sl = pl.ds(h * L, L)
                q = rows_v[t, sl]
                if final:
                    o_v[t, sl] = (x_v[t, sl] - r_v[t, sl]) + q
                else:
                    o_v[t, sl] = r_v[t, sl] - q
            return 0

        lax.fori_loop(0, CHUNK, _tok, 0)
        pltpu.sync_copy(o_v, o_hbm.at[pl.ds(base, CHUNK)])


@functools.lru_cache(maxsize=2)
def _sc_call(final):
    mesh = plsc.VectorSubcoreMesh(
        core_axis_name="c", subcore_axis_name="s",
        num_cores=NC, num_subcores=NS)
    return functools.partial(
        pl.kernel,
        out_type=jax.ShapeDtypeStruct((T, D), jnp.float32),
        mesh=mesh,
        scratch_types=[
            pltpu.VMEM((NCHUNK, CHUNK), jnp.int32),
            pltpu.VMEM((CHUNK, DP), jnp.float32),
            pltpu.VMEM((CHUNK, D), jnp.float32),
            pltpu.VMEM((CHUNK, D), jnp.float32),
            pltpu.VMEM((CHUNK, D), jnp.float32),
            pltpu.SemaphoreType.DMA,
        ],
    )(functools.partial(_sc_body, final))


def kernel(x, codebooks):
    xf = x.reshape(T, D)
    cbt = jnp.transpose(codebooks, (0, 2, 1))
    cbp = jnp.pad(codebooks, ((0, 0), (0, 0), (0, DP - D)))

    inds = []
    stats = []
    r = xf
    for i in range(NUM_Q):
        r2 = jnp.sum(r ** 2, axis=1, keepdims=True)
        ind3, stat = _layer_call(r, r2, cbt[i])
        ind_flat = ind3.reshape(T)
        idx = ind_flat.reshape(NW, NCHUNK, CHUNK)
        if i < NUM_Q - 1:
            r = _sc_call(False)(cbp[i], idx, r, r)
        else:
            out = _sc_call(True)(cbp[i], idx, r, xf)
        inds.append(ind_flat)
        stats.append(stat[0, 0])

    quantized = out.reshape(B, N, D)
    indices = jnp.stack(inds).reshape(NUM_Q, B, N)
    losses = jnp.stack(stats) / float(T * D)
    return quantized, indices, losses
